# baseline (speedup 1.0000x reference)
import functools

import jax
import jax.numpy as jnp
from jax.experimental import pallas as pl
from jax.experimental.pallas import tpu as pltpu


def _gcn_kernel(x_ref, sup_ref, w_ref, b_ref, o_ref, B_ref, *,
                C, V, S, order, BN):
    Cout, TL = o_ref.shape[1], o_ref.shape[3]
    CV = C * V

    @pl.when(pl.program_id(0) == 0)
    def _build_folded_weight():
        rows = jax.lax.broadcasted_iota(jnp.int32, (V, V), 0)
        cols = jax.lax.broadcasted_iota(jnp.int32, (V, V), 1)
        mats = [(rows == cols).astype(jnp.float32)]
        for s in range(S):
            a_t = sup_ref[s].T
            m_k = mats[0]
            for _ in range(order):
                m_k = jnp.dot(a_t, m_k, preferred_element_type=jnp.float32)
                mats.append(m_k)
        for c in range(C):
            acc = None
            for blk, m in enumerate(mats):
                wcol = w_ref[:, blk * C + c][:, None, None]
                term = wcol * m[None, :, :]
                acc = term if acc is None else acc + term
            B_ref[:, :, c * V:(c + 1) * V] = acc.astype(B_ref.dtype)

    Bm = B_ref[...]
    for j in range(BN):
        xj = x_ref[j].reshape(CV, TL).astype(jnp.bfloat16)
        acc = jax.lax.dot_general(Bm, xj, (((2,), (0,)), ((), ())),
                                  preferred_element_type=jnp.float32)
        acc = acc + b_ref[...][:, :, None]
        o_ref[j] = acc.astype(o_ref.dtype)


def kernel(x, support, W, b):
    N, C, V, L = x.shape
    S = support.shape[0]
    Cout, Ctot = W.shape[0], W.shape[1]
    order = (Ctot // C - 1) // S
    CV = C * V

    w2 = W.reshape(Cout, Ctot).astype(jnp.float32)
    b2 = b.reshape(Cout, 1).astype(jnp.float32)

    BN = 8 if N % 8 == 0 else 1
    T = N // BN

    flops = 2 * (Cout * V) * CV * N * L
    bytes_accessed = 4 * (N * C * V * L + N * Cout * V * L)

    kernel_fn = functools.partial(_gcn_kernel, C=C, V=V, S=S, order=order,
                                  BN=BN)
    out = pl.pallas_call(
        kernel_fn,
        out_shape=jax.ShapeDtypeStruct((N, Cout, V, L), x.dtype),
        grid=(T,),
        in_specs=[
            pl.BlockSpec((BN, C, V, L), lambda t: (t, 0, 0, 0)),
            pl.BlockSpec((S, V, V), lambda t: (0, 0, 0)),
            pl.BlockSpec((Cout, Ctot), lambda t: (0, 0)),
            pl.BlockSpec((Cout, 1), lambda t: (0, 0)),
        ],
        out_specs=pl.BlockSpec((BN, Cout, V, L), lambda t: (t, 0, 0, 0)),
        scratch_shapes=[pltpu.VMEM((Cout, V, CV), jnp.bfloat16)],
        compiler_params=pltpu.CompilerParams(
            dimension_semantics=("arbitrary",)),
        cost_estimate=pl.CostEstimate(flops=int(flops), transcendentals=0,
                                      bytes_accessed=int(bytes_accessed)),
    )(x, support.astype(jnp.float32), w2, b2)
    return out


# speedup vs baseline: 1.1692x; 1.1692x over previous
import functools

import jax
import jax.numpy as jnp
from jax.experimental import pallas as pl
from jax.experimental.pallas import tpu as pltpu


def _gcn_kernel(x_ref, sup_ref, w_ref, b_ref, o_ref, B_ref, *,
                C, V, S, order, BN):
    Cout, TL = o_ref.shape[1], o_ref.shape[3]
    CV = C * V

    @pl.when(pl.program_id(0) == 0)
    def _build_folded_weight():
        rows = jax.lax.broadcasted_iota(jnp.int32, (V, V), 0)
        cols = jax.lax.broadcasted_iota(jnp.int32, (V, V), 1)
        mats = [(rows == cols).astype(jnp.float32)]
        for s in range(S):
            a_t = sup_ref[s].T
            m_k = mats[0]
            for _ in range(order):
                m_k = jnp.dot(a_t, m_k, preferred_element_type=jnp.float32)
                mats.append(m_k)
        qc = jax.lax.broadcasted_iota(jnp.int32, (C, CV), 1) // V
        Rw = (jax.lax.broadcasted_iota(jnp.int32, (C, CV), 0)
              == qc).astype(jnp.float32)
        qw = jax.lax.broadcasted_iota(jnp.int32, (V, CV), 1) % V
        Rm = (jax.lax.broadcasted_iota(jnp.int32, (V, CV), 0)
              == qw).astype(jnp.float32)
        total = None
        for blk, m in enumerate(mats):
            w_lane = jnp.dot(w_ref[:, blk * C:(blk + 1) * C], Rw,
                             preferred_element_type=jnp.float32)
            m_lane = Rm if blk == 0 else jnp.dot(
                m, Rm, preferred_element_type=jnp.float32)
            term = w_lane[:, None, :] * m_lane[None, :, :]
            total = term if total is None else total + term
        B_ref[...] = total.astype(B_ref.dtype)

    Bm = B_ref[...]
    for j in range(BN):
        xj = x_ref[j].reshape(CV, TL).astype(jnp.bfloat16)
        acc = jax.lax.dot_general(Bm, xj, (((2,), (0,)), ((), ())),
                                  preferred_element_type=jnp.float32)
        acc = acc + b_ref[...][:, :, None]
        o_ref[j] = acc.astype(o_ref.dtype)


def kernel(x, support, W, b):
    N, C, V, L = x.shape
    S = support.shape[0]
    Cout, Ctot = W.shape[0], W.shape[1]
    order = (Ctot // C - 1) // S
    CV = C * V

    w2 = W.reshape(Cout, Ctot).astype(jnp.float32)
    b2 = b.reshape(Cout, 1).astype(jnp.float32)

    BN = 8 if N % 8 == 0 else 1
    T = N // BN

    flops = 2 * (Cout * V) * CV * N * L
    bytes_accessed = 4 * (N * C * V * L + N * Cout * V * L)

    kernel_fn = functools.partial(_gcn_kernel, C=C, V=V, S=S, order=order,
                                  BN=BN)
    out = pl.pallas_call(
        kernel_fn,
        out_shape=jax.ShapeDtypeStruct((N, Cout, V, L), x.dtype),
        grid=(T,),
        in_specs=[
            pl.BlockSpec((BN, C, V, L), lambda t: (t, 0, 0, 0)),
            pl.BlockSpec((S, V, V), lambda t: (0, 0, 0)),
            pl.BlockSpec((Cout, Ctot), lambda t: (0, 0)),
            pl.BlockSpec((Cout, 1), lambda t: (0, 0)),
        ],
        out_specs=pl.BlockSpec((BN, Cout, V, L), lambda t: (t, 0, 0, 0)),
        scratch_shapes=[pltpu.VMEM((Cout, V, CV), jnp.bfloat16)],
        compiler_params=pltpu.CompilerParams(
            dimension_semantics=("arbitrary",)),
        cost_estimate=pl.CostEstimate(flops=int(flops), transcendentals=0,
                                      bytes_accessed=int(bytes_accessed)),
    )(x, support.astype(jnp.float32), w2, b2)
    return out
